# baseline (speedup 1.0000x reference)
import functools
import math

import jax
import jax.numpy as jnp
from jax import lax
from jax.experimental import pallas as pl
from jax.experimental.pallas import tpu as pltpu
from jax.experimental.pallas import tpu_sc as plsc

D_MODEL = 128
PE_ROWS = 1000
CHUNK = 128
LANES = 16
NBUF = 5
TC_BLOCK = 1024
SC_FRACTION_NUM = 5


def _pe_lookup_body(boxes_hbm, pe_hbm, out_hbm, *scr, rows_per_worker):
    rx = scr[0:NBUF]
    bv = scr[NBUF:2 * NBUF]
    xidx = scr[2 * NBUF:3 * NBUF]
    yidx = scr[3 * NBUF:4 * NBUF]
    pe_sh = scr[4 * NBUF]
    semx = scr[4 * NBUF + 1:5 * NBUF + 1]
    semy = scr[5 * NBUF + 1:6 * NBUF + 1]
    semo = scr[6 * NBUF + 1:7 * NBUF + 1]
    semb = scr[7 * NBUF + 1:8 * NBUF + 1]

    nc = 2
    sid = lax.axis_index("s")
    wid = sid * nc + lax.axis_index("c")
    chunks = rows_per_worker // CHUNK
    row0 = wid * rows_per_worker

    @pl.when(sid == 0)
    def _stage_pe():
        pltpu.sync_copy(pe_hbm, pe_sh)

    plsc.subcore_barrier()

    max_idx = jnp.float32(PE_ROWS - 1)
    lo = jnp.zeros((LANES,), jnp.int32)
    hi = jnp.full((LANES,), PE_ROWS - 1, jnp.int32)

    def start_boxes(t, j):
        return pltpu.async_copy(
            boxes_hbm.at[:, pl.ds(row0 + t * CHUNK, CHUNK)], bv[j], semb[j])

    def wait_boxes(j):
        pltpu.make_async_copy(
            boxes_hbm.at[:, pl.ds(0, CHUNK)], bv[j], semb[j]).wait()

    def compute_idx(j):
        for i in range(CHUNK // LANES):
            sl = pl.ds(i * LANES, LANES)
            b0 = bv[j][0, sl]
            b1 = bv[j][1, sl]
            b2 = bv[j][2, sl]
            b3 = bv[j][3, sl]
            ix = (((b0 + b2) * jnp.float32(0.5)) * max_idx).astype(jnp.int32)
            iy = (((b1 + b3) * jnp.float32(0.5)) * max_idx).astype(jnp.int32)
            xidx[j][sl] = jnp.minimum(jnp.maximum(ix, lo), hi)
            yidx[j][sl] = jnp.minimum(jnp.maximum(iy, lo), hi)

    def start_gather_x(j):
        return pltpu.async_copy(pe_sh.at[xidx[j]], rx[j], semx[j])

    def wait_gather_x(j):
        pltpu.make_async_copy(pe_sh.at[xidx[j]], rx[j], semx[j]).wait()

    for k in range(4):
        start_boxes(k, k)
    wait_boxes(0)
    compute_idx(0)
    wait_boxes(1)
    compute_idx(1)
    start_gather_x(0)

    def outer(g, carry):
        for j in range(NBUF):
            t = g * NBUF + j

            wait_gather_x(j)
            cy = pltpu.async_copy(pe_sh.at[yidx[j]], rx[j], semy[j], add=True)

            jn = (j + 1) % NBUF

            @pl.when(t + 1 < chunks)
            def _next_x():
                @pl.when(t + 1 >= NBUF)
                def _free():
                    pltpu.make_async_copy(
                        rx[jn], out_hbm.at[pl.ds(0, CHUNK)], semo[jn]).wait()
                start_gather_x(jn)

            j2 = (j + 2) % NBUF
            j4 = (j + 4) % NBUF

            @pl.when(t + 2 < chunks)
            def _ahead():
                wait_boxes(j2)
                compute_idx(j2)

                @pl.when(t + 4 < chunks)
                def _pref():
                    start_boxes(t + 4, j4)

            cy.wait()
            pltpu.async_copy(
                rx[j], out_hbm.at[pl.ds(row0 + t * CHUNK, CHUNK)], semo[j])
        return carry

    lax.fori_loop(0, chunks // NBUF, outer, 0)

    for j in range(NBUF):
        pltpu.make_async_copy(
            rx[j], out_hbm.at[pl.ds(0, CHUNK)], semo[j]).wait()


def _tc_body(boxes_ref, freq_ref, phase_ref, prev_ref, out_ref):
    del prev_ref
    b = boxes_ref[...]
    max_idx = jnp.float32(PE_ROWS - 1)
    xi = (((b[:, 0:1] + b[:, 2:3]) * jnp.float32(0.5)) * max_idx)
    yi = (((b[:, 1:2] + b[:, 3:4]) * jnp.float32(0.5)) * max_idx)
    xf = xi.astype(jnp.int32).astype(jnp.float32)
    yf = yi.astype(jnp.int32).astype(jnp.float32)
    f = freq_ref[...]
    ph = phase_ref[...]
    out_ref[...] = jnp.sin(xf * f + ph) + jnp.sin(yf * f + ph)


def _tc_encode(boxes_rows, out_sc, n_sc):
    m = boxes_rows.shape[0]
    n = out_sc.shape[0]
    half = jnp.exp(jnp.arange(0, D_MODEL, 2, dtype=jnp.float32)
                   * jnp.float32(-math.log(10000.0) / D_MODEL))
    freq = jnp.repeat(half, 2).reshape(1, D_MODEL)
    phase = jnp.tile(jnp.array([0.0, math.pi / 2], jnp.float32),
                     D_MODEL // 2).reshape(1, D_MODEL)
    base_blk = n_sc // TC_BLOCK
    return pl.pallas_call(
        _tc_body,
        grid=(m // TC_BLOCK,),
        in_specs=[
            pl.BlockSpec((TC_BLOCK, 4), lambda i: (i, 0)),
            pl.BlockSpec((1, D_MODEL), lambda i: (0, 0)),
            pl.BlockSpec((1, D_MODEL), lambda i: (0, 0)),
            pl.BlockSpec(memory_space=pl.ANY),
        ],
        out_specs=pl.BlockSpec((TC_BLOCK, D_MODEL),
                               lambda i: (i + base_blk, 0)),
        out_shape=jax.ShapeDtypeStruct((n, D_MODEL), jnp.float32),
        input_output_aliases={3: 0},
    )(boxes_rows, freq, phase, out_sc)


def kernel(boxes, pe):
    bsz, seq, _ = boxes.shape
    n = bsz * seq
    num_workers = 32
    n_sc = (n * SC_FRACTION_NUM // 10) // (num_workers * CHUNK * NBUF) \
        * (num_workers * CHUNK * NBUF)
    n_tc = n - n_sc
    assert n_tc % TC_BLOCK == 0
    rows_per_worker = n_sc // num_workers
    chunks = rows_per_worker // CHUNK
    assert chunks % NBUF == 0 and chunks >= 2 * NBUF

    boxes_flat = boxes.reshape(n, 4)
    mesh = plsc.VectorSubcoreMesh(core_axis_name="c", subcore_axis_name="s")
    k = pl.kernel(
        functools.partial(_pe_lookup_body, rows_per_worker=rows_per_worker),
        out_type=jax.ShapeDtypeStruct((n, D_MODEL), jnp.float32),
        mesh=mesh,
        scratch_types=(
            [pltpu.VMEM((CHUNK, D_MODEL), jnp.float32)] * NBUF
            + [pltpu.VMEM((4, CHUNK), jnp.float32)] * NBUF
            + [pltpu.VMEM((CHUNK,), jnp.int32)] * NBUF
            + [pltpu.VMEM((CHUNK,), jnp.int32)] * NBUF
            + [pltpu.VMEM_SHARED((PE_ROWS, D_MODEL), jnp.float32)]
            + [pltpu.SemaphoreType.DMA] * (4 * NBUF)
        ),
    )
    out_sc = k(jnp.transpose(boxes_flat[:n_sc]), pe)
    if n_tc:
        out = _tc_encode(boxes_flat[n_sc:], out_sc, n_sc)
    else:
        out = out_sc
    return out.reshape(bsz, seq, D_MODEL)


# speedup vs baseline: 4.2870x; 4.2870x over previous
import functools
import math

import jax
import jax.numpy as jnp
from jax import lax
from jax.experimental import pallas as pl
from jax.experimental.pallas import tpu as pltpu
from jax.experimental.pallas import tpu_sc as plsc

D_MODEL = 128
PE_ROWS = 1000
CHUNK = 128
LANES = 16
NBUF = 5
TC_BLOCK = 1024
SC_FRACTION_NUM = 10


def _pe_lookup_body(boxes_hbm, pe_hbm, out_hbm, *scr, rows_per_worker):
    rx = scr[0:NBUF]
    bv = scr[NBUF:2 * NBUF]
    xidx = scr[2 * NBUF:3 * NBUF]
    yidx = scr[3 * NBUF:4 * NBUF]
    pe_sh = scr[4 * NBUF]
    semx = scr[4 * NBUF + 1:5 * NBUF + 1]
    semy = scr[5 * NBUF + 1:6 * NBUF + 1]
    semo = scr[6 * NBUF + 1:7 * NBUF + 1]
    semb = scr[7 * NBUF + 1:8 * NBUF + 1]

    nc = 2
    sid = lax.axis_index("s")
    wid = sid * nc + lax.axis_index("c")
    chunks = rows_per_worker // CHUNK
    row0 = wid * rows_per_worker

    @pl.when(sid == 0)
    def _stage_pe():
        pltpu.sync_copy(pe_hbm, pe_sh)

    plsc.subcore_barrier()

    max_idx = jnp.float32(PE_ROWS - 1)
    lo = jnp.zeros((LANES,), jnp.int32)
    hi = jnp.full((LANES,), PE_ROWS - 1, jnp.int32)

    def start_boxes(t, j):
        return pltpu.async_copy(
            boxes_hbm.at[:, pl.ds(row0 + t * CHUNK, CHUNK)], bv[j], semb[j])

    def wait_boxes(j):
        pltpu.make_async_copy(
            boxes_hbm.at[:, pl.ds(0, CHUNK)], bv[j], semb[j]).wait()

    def compute_idx(j):
        for i in range(CHUNK // LANES):
            sl = pl.ds(i * LANES, LANES)
            b0 = bv[j][0, sl]
            b1 = bv[j][1, sl]
            b2 = bv[j][2, sl]
            b3 = bv[j][3, sl]
            ix = (((b0 + b2) * jnp.float32(0.5)) * max_idx).astype(jnp.int32)
            iy = (((b1 + b3) * jnp.float32(0.5)) * max_idx).astype(jnp.int32)
            xidx[j][sl] = jnp.minimum(jnp.maximum(ix, lo), hi)
            yidx[j][sl] = jnp.minimum(jnp.maximum(iy, lo), hi)

    def start_gather_x(j):
        return pltpu.async_copy(pe_sh.at[xidx[j]], rx[j], semx[j])

    def wait_gather_x(j):
        pltpu.make_async_copy(pe_sh.at[xidx[j]], rx[j], semx[j]).wait()

    for k in range(4):
        start_boxes(k, k)
    wait_boxes(0)
    compute_idx(0)
    wait_boxes(1)
    compute_idx(1)
    start_gather_x(0)

    def outer(g, carry):
        for j in range(NBUF):
            t = g * NBUF + j

            wait_gather_x(j)
            cy = pltpu.async_copy(pe_sh.at[yidx[j]], rx[j], semy[j], add=True)

            jn = (j + 1) % NBUF

            @pl.when(t + 1 < chunks)
            def _next_x():
                @pl.when(t + 1 >= NBUF)
                def _free():
                    pltpu.make_async_copy(
                        rx[jn], out_hbm.at[pl.ds(0, CHUNK)], semo[jn]).wait()
                start_gather_x(jn)

            j2 = (j + 2) % NBUF
            j4 = (j + 4) % NBUF

            @pl.when(t + 2 < chunks)
            def _ahead():
                wait_boxes(j2)
                compute_idx(j2)

                @pl.when(t + 4 < chunks)
                def _pref():
                    start_boxes(t + 4, j4)

            cy.wait()
            pltpu.async_copy(
                rx[j], out_hbm.at[pl.ds(row0 + t * CHUNK, CHUNK)], semo[j])
        return carry

    lax.fori_loop(0, chunks // NBUF, outer, 0)

    for j in range(NBUF):
        pltpu.make_async_copy(
            rx[j], out_hbm.at[pl.ds(0, CHUNK)], semo[j]).wait()


def _tc_body(boxes_ref, freq_ref, phase_ref, prev_ref, out_ref):
    del prev_ref
    b = boxes_ref[...]
    max_idx = jnp.float32(PE_ROWS - 1)
    xi = (((b[:, 0:1] + b[:, 2:3]) * jnp.float32(0.5)) * max_idx)
    yi = (((b[:, 1:2] + b[:, 3:4]) * jnp.float32(0.5)) * max_idx)
    xf = xi.astype(jnp.int32).astype(jnp.float32)
    yf = yi.astype(jnp.int32).astype(jnp.float32)
    f = freq_ref[...]
    ph = phase_ref[...]
    out_ref[...] = jnp.sin(xf * f + ph) + jnp.sin(yf * f + ph)


def _tc_encode(boxes_rows, out_sc, n_sc):
    m = boxes_rows.shape[0]
    n = out_sc.shape[0]
    half = jnp.exp(jnp.arange(0, D_MODEL, 2, dtype=jnp.float32)
                   * jnp.float32(-math.log(10000.0) / D_MODEL))
    freq = jnp.repeat(half, 2).reshape(1, D_MODEL)
    phase = jnp.tile(jnp.array([0.0, math.pi / 2], jnp.float32),
                     D_MODEL // 2).reshape(1, D_MODEL)
    base_blk = n_sc // TC_BLOCK
    return pl.pallas_call(
        _tc_body,
        grid=(m // TC_BLOCK,),
        in_specs=[
            pl.BlockSpec((TC_BLOCK, 4), lambda i: (i, 0)),
            pl.BlockSpec((1, D_MODEL), lambda i: (0, 0)),
            pl.BlockSpec((1, D_MODEL), lambda i: (0, 0)),
            pl.BlockSpec(memory_space=pl.ANY),
        ],
        out_specs=pl.BlockSpec((TC_BLOCK, D_MODEL),
                               lambda i: (i + base_blk, 0)),
        out_shape=jax.ShapeDtypeStruct((n, D_MODEL), jnp.float32),
        input_output_aliases={3: 0},
    )(boxes_rows, freq, phase, out_sc)


def kernel(boxes, pe):
    bsz, seq, _ = boxes.shape
    n = bsz * seq
    num_workers = 32
    n_sc = (n * SC_FRACTION_NUM // 10) // (num_workers * CHUNK * NBUF) \
        * (num_workers * CHUNK * NBUF)
    n_tc = n - n_sc
    assert n_tc % TC_BLOCK == 0
    rows_per_worker = n_sc // num_workers
    chunks = rows_per_worker // CHUNK
    assert chunks % NBUF == 0 and chunks >= 2 * NBUF

    boxes_flat = boxes.reshape(n, 4)
    boxes_planes = boxes.transpose(2, 0, 1).reshape(4, n)
    mesh = plsc.VectorSubcoreMesh(core_axis_name="c", subcore_axis_name="s")
    k = pl.kernel(
        functools.partial(_pe_lookup_body, rows_per_worker=rows_per_worker),
        out_type=jax.ShapeDtypeStruct((n, D_MODEL), jnp.float32),
        mesh=mesh,
        scratch_types=(
            [pltpu.VMEM((CHUNK, D_MODEL), jnp.float32)] * NBUF
            + [pltpu.VMEM((4, CHUNK), jnp.float32)] * NBUF
            + [pltpu.VMEM((CHUNK,), jnp.int32)] * NBUF
            + [pltpu.VMEM((CHUNK,), jnp.int32)] * NBUF
            + [pltpu.VMEM_SHARED((PE_ROWS, D_MODEL), jnp.float32)]
            + [pltpu.SemaphoreType.DMA] * (4 * NBUF)
        ),
    )
    out_sc = k(boxes_planes[:, :n_sc], pe)
    if n_tc:
        out = _tc_encode(boxes_flat[n_sc:], out_sc, n_sc)
    else:
        out = out_sc
    return out.reshape(bsz, seq, D_MODEL)
